# baseline (speedup 1.0000x reference)
import functools

import jax
import jax.numpy as jnp
from jax import lax
from jax.experimental import pallas as pl
from jax.experimental.pallas import tpu as pltpu
from jax.experimental.pallas import tpu_sc as plsc

_TOKENS = 1000000
_WORDS = 200
_COORDS = 32
_BATCH = 4096

_NW = 32
_SEQ_PER_W = _BATCH // _NW
_WPC = 2
_NCHUNK = _WORDS // _WPC
_NBUF = 4
_LOOKAHEAD = 2
_TILE = 8 * 128
_WTILE = _COORDS * _SEQ_PER_W


def _encoder(seq_bw, vocab_table, pos_table):
    mesh = plsc.VectorSubcoreMesh(core_axis_name="c", subcore_axis_name="s")

    @functools.partial(
        pl.kernel,
        mesh=mesh,
        out_type=jax.ShapeDtypeStruct((_WORDS * 4 * _NW * _TILE,),
                                      jnp.float32),
        scratch_types=[
            pltpu.VMEM((_WORDS, _SEQ_PER_W), jnp.int32),
            pltpu.VMEM((_NBUF, _WPC, _SEQ_PER_W, _COORDS), jnp.float32),
            pltpu.VMEM((_NBUF * _WPC * _WTILE,), jnp.float32),
            pltpu.VMEM((_WORDS, _COORDS), jnp.float32),
        ]
        + [pltpu.SemaphoreType.DMA] * (2 * _NBUF),
        compiler_params=pltpu.CompilerParams(
            use_tc_tiling_on_sc=False, needs_layout_passes=False),
    )
    def body(seq_hbm, vocab_hbm, pos_hbm, out_hbm, idxT_v, rows_v,
             stage_v, pos_v, *sems):
        wid = lax.axis_index("s") * 2 + lax.axis_index("c")
        base = wid * _SEQ_PER_W
        gsems = sems[:_NBUF]
        osems = sems[_NBUF:]

        pltpu.sync_copy(pos_hbm, pos_v)

        iota = lax.iota(jnp.int32, 16)

        pltpu.sync_copy(seq_hbm.at[pl.ds(0, _WORDS), pl.ds(base, _SEQ_PER_W)],
                        idxT_v)

        def fire(c):
            buf = c % _NBUF
            return [
                pltpu.async_copy(
                    vocab_hbm.at[idxT_v.at[c * _WPC + k]],
                    rows_v.at[buf, k],
                    gsems[buf],
                )
                for k in range(_WPC)
            ]

        gathers = {c: fire(c) for c in range(min(_LOOKAHEAD, _NCHUNK))}
        writebacks = {}
        for c in range(_NCHUNK):
            buf = c % _NBUF
            nxt = c + _LOOKAHEAD
            if nxt < _NCHUNK:
                gathers[nxt] = fire(nxt)
            for cp in gathers.pop(c):
                cp.wait()
            if c - _NBUF in writebacks:
                for wb in writebacks.pop(c - _NBUF):
                    wb.wait()

            sc_base = buf * _WPC * _WTILE + iota * _SEQ_PER_W

            @pl.loop(0, _WPC * _SEQ_PER_W)
            def _(r):
                k = r >> 7
                b = r & 127
                w = c * _WPC + k
                p0 = pos_v[w, pl.ds(0, 16)]
                p1 = pos_v[w, pl.ds(16, 16)]
                s0 = rows_v[buf, k, b, pl.ds(0, 16)] + p0
                s1 = rows_v[buf, k, b, pl.ds(16, 16)] + p1
                sc0 = sc_base + k * _WTILE + b
                plsc.store_scatter(stage_v, [sc0], s0)
                plsc.store_scatter(stage_v, [sc0 + 16 * _SEQ_PER_W], s1)

            writebacks[c] = [
                pltpu.async_copy(
                    stage_v.at[pl.ds((buf * _WPC + k) * _WTILE + tr * _TILE,
                                     _TILE)],
                    out_hbm.at[pl.ds(
                        (((c * _WPC + k) * 4 + tr) * _NW + wid) * _TILE,
                        _TILE)],
                    osems[buf],
                )
                for k in range(_WPC)
                for tr in range(4)
            ]
        for wbs in writebacks.values():
            for wb in wbs:
                wb.wait()

    return body(seq_bw, vocab_table, pos_table)


def kernel(sequence_bw, vocab_table, pos_table):
    seq_t = sequence_bw.astype(jnp.int32).T
    out1d = _encoder(seq_t, vocab_table, pos_table)
    out5d = out1d.reshape(_WORDS, 4, _NW, 8, 128)
    return out5d.transpose(2, 4, 0, 1, 3).reshape(_BATCH, _WORDS, _COORDS)


# speedup vs baseline: 1.0778x; 1.0778x over previous
import functools

import jax
import jax.numpy as jnp
from jax import lax
from jax.experimental import pallas as pl
from jax.experimental.pallas import tpu as pltpu
from jax.experimental.pallas import tpu_sc as plsc

_TOKENS = 1000000
_WORDS = 200
_COORDS = 32
_BATCH = 4096

_NW = 32
_SEQ_PER_W = _BATCH // _NW
_SPC = 2
_NCHUNK = _SEQ_PER_W // _SPC
_SPLITS = ((0, 96), (96, 104))
_NBUF = 5
_LOOKAHEAD = 3


def _encoder(seq_bw, vocab_table, pos_table):
    mesh = plsc.VectorSubcoreMesh(core_axis_name="c", subcore_axis_name="s")

    @functools.partial(
        pl.kernel,
        mesh=mesh,
        out_type=jax.ShapeDtypeStruct((_BATCH, _WORDS, _COORDS), jnp.float32),
        scratch_types=[
            pltpu.VMEM((_SEQ_PER_W, _WORDS), jnp.int32),
            pltpu.VMEM((_NBUF, _SPC, _WORDS, _COORDS), jnp.float32),
            pltpu.VMEM((_WORDS, _COORDS), jnp.float32),
        ]
        + [pltpu.SemaphoreType.DMA] * (2 * _NBUF),
        compiler_params=pltpu.CompilerParams(use_tc_tiling_on_sc=False),
    )
    def body(seq_hbm, vocab_hbm, pos_hbm, out_hbm, idx_v, rows_v, pos_v,
             *sems):
        wid = lax.axis_index("s") * 2 + lax.axis_index("c")
        base = wid * _SEQ_PER_W
        gsems = sems[:_NBUF]
        osems = sems[_NBUF:]

        pltpu.sync_copy(seq_hbm.at[pl.ds(base, _SEQ_PER_W)], idx_v)
        pltpu.sync_copy(pos_hbm, pos_v)

        def fire(c):
            buf = c % _NBUF
            return [
                pltpu.async_copy(
                    vocab_hbm.at[idx_v.at[c * _SPC + k, pl.ds(off, ln)]],
                    rows_v.at[buf, k, pl.ds(off, ln)],
                    gsems[buf],
                )
                for k in range(_SPC)
                for off, ln in _SPLITS
            ]

        gathers = {c: fire(c) for c in range(min(_LOOKAHEAD, _NCHUNK))}
        writebacks = {}
        for c in range(_NCHUNK):
            buf = c % _NBUF
            nxt = c + _LOOKAHEAD
            if nxt < _NCHUNK:
                if nxt - _NBUF in writebacks:
                    writebacks.pop(nxt - _NBUF).wait()
                gathers[nxt] = fire(nxt)
            for cp in gathers.pop(c):
                cp.wait()

            @pl.loop(0, _WORDS)
            def _(r):
                v0 = pos_v[r, pl.ds(0, 16)]
                v1 = pos_v[r, pl.ds(16, 16)]
                for k in range(_SPC):
                    plsc.addupdate(rows_v.at[buf, k, r, pl.ds(0, 16)], v0)
                    plsc.addupdate(rows_v.at[buf, k, r, pl.ds(16, 16)], v1)

            writebacks[c] = pltpu.async_copy(
                rows_v.at[buf], out_hbm.at[pl.ds(base + c * _SPC, _SPC)],
                osems[buf],
            )
        for wb in writebacks.values():
            wb.wait()

    return body(seq_bw, vocab_table, pos_table)


def kernel(sequence_bw, vocab_table, pos_table):
    return _encoder(sequence_bw.astype(jnp.int32), vocab_table, pos_table)
